# baseline (speedup 1.0000x reference)
import functools

import jax
import jax.numpy as jnp
from jax import lax
from jax.experimental import pallas as pl
from jax.experimental.pallas import tpu as pltpu
from jax.experimental.pallas import tpu_sc as plsc

NC = 2
NS = 16
NW = NC * NS
IDX_W = 128
PAD_W = 128


def _tpad_call(vocab, dim):
    blk = 8192

    def body(tt_ref, o_ref):
        x = tt_ref[...].T
        o_ref[:, 0:dim] = x

    return pl.pallas_call(
        body,
        grid=(pl.cdiv(vocab, blk),),
        in_specs=[pl.BlockSpec((dim, blk), lambda i: (0, i))],
        out_specs=pl.BlockSpec((blk, PAD_W), lambda i: (i, 0)),
        out_shape=jax.ShapeDtypeStruct((vocab, PAD_W), jnp.float32),
    )


def _gather_call(batch, seq, dim, nbuf):
    per_worker = batch // NW
    assert batch % NW == 0
    splits = []
    t0 = 0
    while t0 < seq:
        w = min(IDX_W, seq - t0)
        splits.append((t0, w))
        t0 += w

    mesh = plsc.VectorSubcoreMesh(
        core_axis_name="c", subcore_axis_name="s",
        num_cores=NC, num_subcores=NS,
    )

    tail = per_worker % nbuf

    @functools.partial(
        pl.kernel,
        out_type=jax.ShapeDtypeStruct((batch * seq, dim), jnp.float32),
        mesh=mesh,
        scratch_types=[
            [pltpu.VMEM((1, seq), jnp.int32) for _ in range(nbuf)],
            [pltpu.VMEM((seq, PAD_W), jnp.float32) for _ in range(nbuf)],
            pltpu.VMEM((seq, dim), jnp.float32),
            [pltpu.SemaphoreType.DMA for _ in range(nbuf)],
            [pltpu.SemaphoreType.DMA for _ in range(nbuf)],
            pltpu.SemaphoreType.DMA,
        ],
    )
    def gather_kernel(ids_hbm, table_hbm, out_hbm, idx_v, rows_v, cmp_v,
                      isems, gsems, osem):
        wid = lax.axis_index("s") * NC + lax.axis_index("c")
        seq_base = wid * per_worker

        def idx_copy(c, b):
            return pltpu.make_async_copy(
                ids_hbm.at[pl.ds(seq_base + c, 1)], idx_v[b], isems[b])

        def gather_copies(b):
            return [
                pltpu.make_async_copy(
                    table_hbm.at[idx_v[b].at[0, pl.ds(t0, w)]],
                    rows_v[b].at[pl.ds(t0, w)],
                    gsems[b],
                )
                for (t0, w) in splits
            ]

        def wb_copy(c):
            return pltpu.make_async_copy(
                cmp_v,
                out_hbm.at[pl.ds((seq_base + c) * seq, seq)],
                osem)

        for b in range(nbuf):
            idx_copy(b, b).start()
        for b in range(nbuf):
            idx_copy(b, b).wait()
            for cp in gather_copies(b):
                cp.start()

        def slot(c, b):
            for cp in gather_copies(b):
                cp.wait()
            nxt = c + nbuf

            @pl.when(nxt < per_worker)
            def _():
                idx_copy(nxt, b).start()

            @pl.when(c >= 1)
            def _():
                wb_copy(c - 1).wait()

            def crow(r, carry):
                for j in range(dim // 16):
                    cmp_v[r, pl.ds(j * 16, 16)] = (
                        rows_v[b][r, pl.ds(j * 16, 16)])
                return carry
            lax.fori_loop(0, seq, crow, 0, unroll=50)
            wb_copy(c).start()

            @pl.when(nxt < per_worker)
            def _():
                idx_copy(nxt, b).wait()
                for cp in gather_copies(b):
                    cp.start()

        def body(g, carry):
            for b in range(nbuf):
                slot(g * nbuf + b, b)
            return carry

        lax.fori_loop(0, per_worker // nbuf, body, 0, unroll=False)
        for k in range(tail):
            c = per_worker - tail + k
            slot(jnp.int32(c), c % nbuf)
        wb_copy(0).wait()

    return gather_kernel


def kernel(token_ids, embeddings):
    batch, seq = token_ids.shape
    vocab, dim = embeddings.shape

    table_pad = _tpad_call(vocab, dim)(embeddings.T)
    flat = _gather_call(batch, seq, dim, 3)(token_ids, table_pad)
    x = flat.reshape(batch, seq, dim)

    mask = jnp.ones((batch, seq), dtype=jnp.float32)
    time_steps = jnp.array(seq, dtype=jnp.int32)
    return (x, mask, time_steps)


# speedup vs baseline: 1.0050x; 1.0050x over previous
import functools

import jax
import jax.numpy as jnp
from jax import lax
from jax.experimental import pallas as pl
from jax.experimental.pallas import tpu as pltpu
from jax.experimental.pallas import tpu_sc as plsc

NC = 2
NS = 16
NW = NC * NS
IDX_W = 128
PAD_W = 128


def _tpad_call(vocab, dim):
    blk = 8192

    def body(tt_ref, o_ref):
        x = tt_ref[...].T
        o_ref[:, 0:dim] = x

    return pl.pallas_call(
        body,
        grid=(pl.cdiv(vocab, blk),),
        in_specs=[pl.BlockSpec((dim, blk), lambda i: (0, i))],
        out_specs=pl.BlockSpec((blk, PAD_W), lambda i: (i, 0)),
        out_shape=jax.ShapeDtypeStruct((vocab, PAD_W), jnp.float32),
    )


def _gather_call(batch, seq, dim, nbuf):
    per_worker = batch // NW
    assert batch % NW == 0
    splits = []
    t0 = 0
    while t0 < seq:
        w = min(IDX_W, seq - t0)
        splits.append((t0, w))
        t0 += w

    mesh = plsc.VectorSubcoreMesh(
        core_axis_name="c", subcore_axis_name="s",
        num_cores=NC, num_subcores=NS,
    )

    tail = per_worker % nbuf

    @functools.partial(
        pl.kernel,
        out_type=jax.ShapeDtypeStruct((batch * seq, dim), jnp.float32),
        mesh=mesh,
        scratch_types=[
            [pltpu.VMEM((1, seq), jnp.int32) for _ in range(nbuf)],
            [pltpu.VMEM((seq, PAD_W), jnp.float32) for _ in range(nbuf)],
            pltpu.VMEM((seq, dim), jnp.float32),
            [pltpu.SemaphoreType.DMA for _ in range(nbuf)],
            [pltpu.SemaphoreType.DMA for _ in range(nbuf)],
            pltpu.SemaphoreType.DMA,
        ],
    )
    def gather_kernel(ids_hbm, table_hbm, out_hbm, idx_v, rows_v, cmp_v,
                      isems, gsems, osem):
        wid = lax.axis_index("s") * NC + lax.axis_index("c")
        seq_base = wid * per_worker

        def idx_copy(c, b):
            return pltpu.make_async_copy(
                ids_hbm.at[pl.ds(seq_base + c, 1)], idx_v[b], isems[b])

        def gather_copies(b):
            return [
                pltpu.make_async_copy(
                    table_hbm.at[idx_v[b].at[0, pl.ds(t0, w)]],
                    rows_v[b].at[pl.ds(t0, w)],
                    gsems[b],
                )
                for (t0, w) in splits
            ]

        def wb_copy(c):
            return pltpu.make_async_copy(
                cmp_v,
                out_hbm.at[pl.ds((seq_base + c) * seq, seq)],
                osem)

        for b in range(nbuf):
            idx_copy(b, b).start()
        for b in range(nbuf):
            idx_copy(b, b).wait()
            for cp in gather_copies(b):
                cp.start()

        def slot(c, b):
            for cp in gather_copies(b):
                cp.wait()
            nxt = c + nbuf

            @pl.when(nxt < per_worker)
            def _():
                idx_copy(nxt, b).start()

            @pl.when(c >= 1)
            def _():
                wb_copy(c - 1).wait()

            def crow(r, carry):
                for j in range(dim // 16):
                    cmp_v[r, pl.ds(j * 16, 16)] = (
                        rows_v[b][r, pl.ds(j * 16, 16)])
                return carry
            lax.fori_loop(0, seq, crow, 0, unroll=50)
            wb_copy(c).start()

            @pl.when(nxt < per_worker)
            def _():
                idx_copy(nxt, b).wait()
                for cp in gather_copies(b):
                    cp.start()

        def body(g, carry):
            for b in range(nbuf):
                slot(g * nbuf + b, b)
            return carry

        lax.fori_loop(0, per_worker // nbuf, body, 0, unroll=False)
        for k in range(tail):
            c = per_worker - tail + k
            slot(jnp.int32(c), c % nbuf)
        wb_copy(0).wait()

    return gather_kernel


def kernel(token_ids, embeddings):
    batch, seq = token_ids.shape
    vocab, dim = embeddings.shape

    table_pad = _tpad_call(vocab, dim)(embeddings.T)
    flat = _gather_call(batch, seq, dim, 2)(token_ids, table_pad)
    x = flat.reshape(batch, seq, dim)

    mask = jnp.ones((batch, seq), dtype=jnp.float32)
    time_steps = jnp.array(seq, dtype=jnp.int32)
    return (x, mask, time_steps)


# speedup vs baseline: 1.0327x; 1.0276x over previous
import functools

import jax
import jax.numpy as jnp
from jax import lax
from jax.experimental import pallas as pl
from jax.experimental.pallas import tpu as pltpu
from jax.experimental.pallas import tpu_sc as plsc

NC = 2
NS = 16
NW = NC * NS
IDX_W = 128
PAD_W = 128


def _tpad_call(vocab, dim):
    blk = 16384

    def body(tt_ref, o_ref):
        x = tt_ref[...].T
        o_ref[:, 0:dim] = x

    return pl.pallas_call(
        body,
        grid=(pl.cdiv(vocab, blk),),
        in_specs=[pl.BlockSpec((dim, blk), lambda i: (0, i))],
        out_specs=pl.BlockSpec((blk, PAD_W), lambda i: (i, 0)),
        out_shape=jax.ShapeDtypeStruct((vocab, PAD_W), jnp.float32),
    )


def _gather_call(batch, seq, dim, nbuf):
    per_worker = batch // NW
    assert batch % NW == 0
    splits = []
    t0 = 0
    while t0 < seq:
        w = min(IDX_W, seq - t0)
        splits.append((t0, w))
        t0 += w

    mesh = plsc.VectorSubcoreMesh(
        core_axis_name="c", subcore_axis_name="s",
        num_cores=NC, num_subcores=NS,
    )

    tail = per_worker % nbuf

    @functools.partial(
        pl.kernel,
        out_type=jax.ShapeDtypeStruct((batch * seq, dim), jnp.float32),
        mesh=mesh,
        scratch_types=[
            [pltpu.VMEM((1, seq), jnp.int32) for _ in range(nbuf)],
            [pltpu.VMEM((seq, PAD_W), jnp.float32) for _ in range(nbuf)],
            pltpu.VMEM((seq, dim), jnp.float32),
            [pltpu.SemaphoreType.DMA for _ in range(nbuf)],
            [pltpu.SemaphoreType.DMA for _ in range(nbuf)],
            pltpu.SemaphoreType.DMA,
        ],
    )
    def gather_kernel(ids_hbm, table_hbm, out_hbm, idx_v, rows_v, cmp_v,
                      isems, gsems, osem):
        wid = lax.axis_index("s") * NC + lax.axis_index("c")
        seq_base = wid * per_worker

        def idx_copy(c, b):
            return pltpu.make_async_copy(
                ids_hbm.at[pl.ds(seq_base + c, 1)], idx_v[b], isems[b])

        def gather_copies(b):
            return [
                pltpu.make_async_copy(
                    table_hbm.at[idx_v[b].at[0, pl.ds(t0, w)]],
                    rows_v[b].at[pl.ds(t0, w)],
                    gsems[b],
                )
                for (t0, w) in splits
            ]

        def wb_copy(c):
            return pltpu.make_async_copy(
                cmp_v,
                out_hbm.at[pl.ds((seq_base + c) * seq, seq)],
                osem)

        for b in range(nbuf):
            idx_copy(b, b).start()
        for b in range(nbuf):
            idx_copy(b, b).wait()
            for cp in gather_copies(b):
                cp.start()

        def slot(c, b):
            for cp in gather_copies(b):
                cp.wait()
            nxt = c + nbuf

            @pl.when(nxt < per_worker)
            def _():
                idx_copy(nxt, b).start()

            @pl.when(c >= 1)
            def _():
                wb_copy(c - 1).wait()

            def crow(r, carry):
                for j in range(dim // 16):
                    cmp_v[r, pl.ds(j * 16, 16)] = (
                        rows_v[b][r, pl.ds(j * 16, 16)])
                return carry
            lax.fori_loop(0, seq, crow, 0, unroll=50)
            wb_copy(c).start()

            @pl.when(nxt < per_worker)
            def _():
                idx_copy(nxt, b).wait()
                for cp in gather_copies(b):
                    cp.start()

        def body(g, carry):
            for b in range(nbuf):
                slot(g * nbuf + b, b)
            return carry

        lax.fori_loop(0, per_worker // nbuf, body, 0, unroll=False)
        for k in range(tail):
            c = per_worker - tail + k
            slot(jnp.int32(c), c % nbuf)
        wb_copy(0).wait()

    return gather_kernel


def kernel(token_ids, embeddings):
    batch, seq = token_ids.shape
    vocab, dim = embeddings.shape

    table_pad = _tpad_call(vocab, dim)(embeddings.T)
    flat = _gather_call(batch, seq, dim, 2)(token_ids, table_pad)
    x = flat.reshape(batch, seq, dim)

    mask = jnp.ones((batch, seq), dtype=jnp.float32)
    time_steps = jnp.array(seq, dtype=jnp.int32)
    return (x, mask, time_steps)


# speedup vs baseline: 1.0394x; 1.0065x over previous
import functools

import jax
import jax.numpy as jnp
from jax import lax
from jax.experimental import pallas as pl
from jax.experimental.pallas import tpu as pltpu
from jax.experimental.pallas import tpu_sc as plsc

NC = 2
NS = 16
NW = NC * NS
IDX_W = 128
PAD_W = 128


def _tpad_call(vocab, dim):
    blk = 32768

    def body(tt_ref, o_ref):
        x = tt_ref[...].T
        o_ref[:, 0:dim] = x

    return pl.pallas_call(
        body,
        grid=(pl.cdiv(vocab, blk),),
        in_specs=[pl.BlockSpec((dim, blk), lambda i: (0, i))],
        out_specs=pl.BlockSpec((blk, PAD_W), lambda i: (i, 0)),
        out_shape=jax.ShapeDtypeStruct((vocab, PAD_W), jnp.float32),
    )


def _gather_call(batch, seq, dim, nbuf):
    per_worker = batch // NW
    assert batch % NW == 0
    splits = []
    t0 = 0
    while t0 < seq:
        w = min(IDX_W, seq - t0)
        splits.append((t0, w))
        t0 += w

    mesh = plsc.VectorSubcoreMesh(
        core_axis_name="c", subcore_axis_name="s",
        num_cores=NC, num_subcores=NS,
    )

    tail = per_worker % nbuf

    @functools.partial(
        pl.kernel,
        out_type=jax.ShapeDtypeStruct((batch * seq, dim), jnp.float32),
        mesh=mesh,
        scratch_types=[
            [pltpu.VMEM((1, seq), jnp.int32) for _ in range(nbuf)],
            [pltpu.VMEM((seq, PAD_W), jnp.float32) for _ in range(nbuf)],
            pltpu.VMEM((seq, dim), jnp.float32),
            [pltpu.SemaphoreType.DMA for _ in range(nbuf)],
            [pltpu.SemaphoreType.DMA for _ in range(nbuf)],
            pltpu.SemaphoreType.DMA,
        ],
    )
    def gather_kernel(ids_hbm, table_hbm, out_hbm, idx_v, rows_v, cmp_v,
                      isems, gsems, osem):
        wid = lax.axis_index("s") * NC + lax.axis_index("c")
        seq_base = wid * per_worker

        def idx_copy(c, b):
            return pltpu.make_async_copy(
                ids_hbm.at[pl.ds(seq_base + c, 1)], idx_v[b], isems[b])

        def gather_copies(b):
            return [
                pltpu.make_async_copy(
                    table_hbm.at[idx_v[b].at[0, pl.ds(t0, w)]],
                    rows_v[b].at[pl.ds(t0, w)],
                    gsems[b],
                )
                for (t0, w) in splits
            ]

        def wb_copy(c):
            return pltpu.make_async_copy(
                cmp_v,
                out_hbm.at[pl.ds((seq_base + c) * seq, seq)],
                osem)

        for b in range(nbuf):
            idx_copy(b, b).start()
        for b in range(nbuf):
            idx_copy(b, b).wait()
            for cp in gather_copies(b):
                cp.start()

        def slot(c, b):
            for cp in gather_copies(b):
                cp.wait()
            nxt = c + nbuf

            @pl.when(nxt < per_worker)
            def _():
                idx_copy(nxt, b).start()

            @pl.when(c >= 1)
            def _():
                wb_copy(c - 1).wait()

            def crow(r, carry):
                for j in range(dim // 16):
                    cmp_v[r, pl.ds(j * 16, 16)] = (
                        rows_v[b][r, pl.ds(j * 16, 16)])
                return carry
            lax.fori_loop(0, seq, crow, 0, unroll=50)
            wb_copy(c).start()

            @pl.when(nxt < per_worker)
            def _():
                idx_copy(nxt, b).wait()
                for cp in gather_copies(b):
                    cp.start()

        def body(g, carry):
            for b in range(nbuf):
                slot(g * nbuf + b, b)
            return carry

        lax.fori_loop(0, per_worker // nbuf, body, 0, unroll=False)
        for k in range(tail):
            c = per_worker - tail + k
            slot(jnp.int32(c), c % nbuf)
        wb_copy(0).wait()

    return gather_kernel


def kernel(token_ids, embeddings):
    batch, seq = token_ids.shape
    vocab, dim = embeddings.shape

    table_pad = _tpad_call(vocab, dim)(embeddings.T)
    flat = _gather_call(batch, seq, dim, 2)(token_ids, table_pad)
    x = flat.reshape(batch, seq, dim)

    mask = jnp.ones((batch, seq), dtype=jnp.float32)
    time_steps = jnp.array(seq, dtype=jnp.int32)
    return (x, mask, time_steps)
